# Initial kernel scaffold; baseline (speedup 1.0000x reference)
import jax
import jax.numpy as jnp
from jax.experimental import pallas as pl


def kernel(x, edge_index, W_gcn, b_gcn, W_pt, b_pt, W_ps, b_ps, W_cls, b_cls):
    raise NotImplementedError("write your pallas kernel here")


# speedup vs baseline: 22.3199x; 22.3199x over previous
import functools

import jax
import jax.numpy as jnp
from jax import lax
from jax.experimental import pallas as pl
from jax.experimental.pallas import tpu as pltpu
from jax.experimental.pallas import tpu_sc as plsc

N = 10000
E = 320000
IN_DIM = 128
HID = 64
ALIGN = 32
NUM_CLASSES = 7

NC = 2
NS = 16
NW = NC * NS

NPAD = 10240
ROWS_PER_TILE = NPAD // NS
CH = 128
CPW = 80
EPW = CPW * CH
EPAD = NW * EPW
GARBAGE = N + 64
NB = 4

BR = 1024
GRID = NPAD // BR


def _sc_mesh():
    return plsc.VectorSubcoreMesh(core_axis_name="c", subcore_axis_name="s")


def _deg_body(dst_hbm, zeros_hbm, out_hbm, idx_v, ones_v, acc_sh, isem, asem):
    c = lax.axis_index("c")
    s = lax.axis_index("s")
    w = s * NC + c
    rslice = pl.ds(s * ROWS_PER_TILE, ROWS_PER_TILE)
    for i in range(8):
        ones_v[pl.ds(i * 16, 16)] = jnp.ones((16,), jnp.float32)
    pltpu.async_copy(zeros_hbm.at[rslice], acc_sh.at[rslice], isem).wait()
    pltpu.sync_copy(dst_hbm.at[pl.ds(w * CPW, CPW)], idx_v)
    plsc.subcore_barrier()

    @pl.loop(0, CPW, step=NB)
    def _chunks(t):
        hs = [
            pltpu.async_copy(ones_v, acc_sh.at[idx_v.at[t + b]], asem, add=True)
            for b in range(NB)
        ]
        for h in hs:
            h.wait()

    plsc.subcore_barrier()
    pltpu.sync_copy(acc_sh.at[rslice], out_hbm.at[c].at[rslice])


def _sc_degree(dst2d, zeros1d):
    return pl.kernel(
        _deg_body,
        out_type=jax.ShapeDtypeStruct((NC, NPAD), jnp.float32),
        mesh=_sc_mesh(),
        scratch_types=[
            pltpu.VMEM((CPW, CH), jnp.int32),
            pltpu.VMEM((CH,), jnp.float32),
            pltpu.VMEM_SHARED((NPAD,), jnp.float32),
            pltpu.SemaphoreType.DMA,
            pltpu.SemaphoreType.DMA,
        ],
    )(dst2d, zeros1d)


def _scat_body(y_hbm, src_hbm, dst_hbm, zeros_hbm, out_hbm,
               src_v, dst_v, rows_v, acc_sh, isem, gsem, ssem):
    c = lax.axis_index("c")
    s = lax.axis_index("s")
    w = s * NC + c
    rslice = pl.ds(s * ROWS_PER_TILE, ROWS_PER_TILE)

    @pl.when(c == 0)
    def _():
        pltpu.async_copy(y_hbm.at[rslice], acc_sh.at[rslice], isem).wait()

    @pl.when(c != 0)
    def _():
        pltpu.async_copy(zeros_hbm.at[rslice], acc_sh.at[rslice], isem).wait()

    pltpu.sync_copy(src_hbm.at[pl.ds(w * CPW, CPW)], src_v)
    pltpu.sync_copy(dst_hbm.at[pl.ds(w * CPW, CPW)], dst_v)
    plsc.subcore_barrier()

    @pl.loop(0, CPW, step=NB)
    def _chunks(t):
        ghs = [
            pltpu.async_copy(y_hbm.at[src_v.at[t + b]], rows_v.at[b], gsem)
            for b in range(NB)
        ]
        for h in ghs:
            h.wait()
        shs = [
            pltpu.async_copy(rows_v.at[b], acc_sh.at[dst_v.at[t + b]], ssem,
                             add=True)
            for b in range(NB)
        ]
        for h in shs:
            h.wait()

    plsc.subcore_barrier()
    pltpu.sync_copy(acc_sh.at[rslice], out_hbm.at[c].at[rslice])


def _sc_scatter(y, src2d, dst2d, zeros2d):
    return pl.kernel(
        _scat_body,
        out_type=jax.ShapeDtypeStruct((NC, NPAD, HID), jnp.float32),
        mesh=_sc_mesh(),
        compiler_params=pltpu.CompilerParams(use_tc_tiling_on_sc=False),
        scratch_types=[
            pltpu.VMEM((CPW, CH), jnp.int32),
            pltpu.VMEM((CPW, CH), jnp.int32),
            pltpu.VMEM((NB, CH, HID), jnp.float32),
            pltpu.VMEM_SHARED((NPAD, HID), jnp.float32),
            pltpu.SemaphoreType.DMA,
            pltpu.SemaphoreType.DMA,
            pltpu.SemaphoreType.DMA,
        ],
    )(y, src2d, dst2d, zeros2d)


def _pre_body(x_ref, degp_ref, wg_ref, wps_ref, bps_ref, y_ref, zsem_ref):
    deg = degp_ref[0] + degp_ref[1] + 1.0
    dinv = lax.rsqrt(deg)
    xb = x_ref[...]
    xw = jnp.dot(xb, wg_ref[...], preferred_element_type=jnp.float32)
    y_ref[...] = xw * dinv[:, None]
    zsem_ref[...] = (
        jnp.dot(xb, wps_ref[...], preferred_element_type=jnp.float32)
        + bps_ref[...]
    )


def _tc_pre(xp, deg_part, W_gcn, W_ps, b_ps2):
    return pl.pallas_call(
        _pre_body,
        grid=(GRID,),
        in_specs=[
            pl.BlockSpec((BR, IN_DIM), lambda i: (i, 0)),
            pl.BlockSpec((NC, BR), lambda i: (0, i)),
            pl.BlockSpec((IN_DIM, HID), lambda i: (0, 0)),
            pl.BlockSpec((IN_DIM, ALIGN), lambda i: (0, 0)),
            pl.BlockSpec((1, ALIGN), lambda i: (0, 0)),
        ],
        out_specs=[
            pl.BlockSpec((BR, HID), lambda i: (i, 0)),
            pl.BlockSpec((BR, ALIGN), lambda i: (i, 0)),
        ],
        out_shape=[
            jax.ShapeDtypeStruct((NPAD, HID), jnp.float32),
            jax.ShapeDtypeStruct((NPAD, ALIGN), jnp.float32),
        ],
    )(xp, deg_part, W_gcn, W_ps, b_ps2)


def _post_body(acc_ref, degp_ref, zsem_ref, wpt_ref, wcls_ref,
               bg_ref, bpt_ref, bcls_ref, zt_ref, o8_ref):
    deg = degp_ref[0] + degp_ref[1] + 1.0
    dinv = lax.rsqrt(deg)
    agg = (acc_ref[0] + acc_ref[1]) * dinv[:, None] + bg_ref[...]
    h = jnp.maximum(agg, 0.0)
    zt = jnp.dot(h, wpt_ref[...], preferred_element_type=jnp.float32) + bpt_ref[...]
    zt_ref[...] = zt
    o8 = jnp.dot(zt, wcls_ref[...], preferred_element_type=jnp.float32) + bcls_ref[...]
    diff = zt - zsem_ref[...]
    an = jnp.sqrt(jnp.sum(diff * diff, axis=-1))
    col = lax.broadcasted_iota(jnp.int32, (BR, 8), 1)
    o8_ref[...] = jnp.where(col == NUM_CLASSES, an[:, None], o8)


def _tc_post(acc_part, deg_part, zsem, W_pt, Wcls8, b_gcn2, b_pt2, bcls8):
    return pl.pallas_call(
        _post_body,
        grid=(GRID,),
        in_specs=[
            pl.BlockSpec((NC, BR, HID), lambda i: (0, i, 0)),
            pl.BlockSpec((NC, BR), lambda i: (0, i)),
            pl.BlockSpec((BR, ALIGN), lambda i: (i, 0)),
            pl.BlockSpec((HID, ALIGN), lambda i: (0, 0)),
            pl.BlockSpec((ALIGN, 8), lambda i: (0, 0)),
            pl.BlockSpec((1, HID), lambda i: (0, 0)),
            pl.BlockSpec((1, ALIGN), lambda i: (0, 0)),
            pl.BlockSpec((1, 8), lambda i: (0, 0)),
        ],
        out_specs=[
            pl.BlockSpec((BR, ALIGN), lambda i: (i, 0)),
            pl.BlockSpec((BR, 8), lambda i: (i, 0)),
        ],
        out_shape=[
            jax.ShapeDtypeStruct((NPAD, ALIGN), jnp.float32),
            jax.ShapeDtypeStruct((NPAD, 8), jnp.float32),
        ],
    )(acc_part, deg_part, zsem, W_pt, Wcls8, b_gcn2, b_pt2, bcls8)


def kernel(x, edge_index, W_gcn, b_gcn, W_pt, b_pt, W_ps, b_ps, W_cls, b_cls):
    f32 = jnp.float32
    pad_e = EPAD - E
    src2d = jnp.concatenate(
        [edge_index[0], jnp.zeros((pad_e,), jnp.int32)]).reshape(EPAD // CH, CH)
    dst2d = jnp.concatenate(
        [edge_index[1], jnp.full((pad_e,), GARBAGE, jnp.int32)]
    ).reshape(EPAD // CH, CH)
    xp = jnp.pad(x, ((0, NPAD - N), (0, 0)))

    deg_part = _sc_degree(dst2d, jnp.zeros((NPAD,), f32))

    y, zsem = _tc_pre(xp, deg_part, W_gcn, W_ps, b_ps.reshape(1, ALIGN))

    acc_part = _sc_scatter(y, src2d, dst2d, jnp.zeros((NPAD, HID), f32))

    Wcls8 = jnp.pad(W_cls, ((0, 0), (0, 8 - NUM_CLASSES)))
    bcls8 = jnp.pad(b_cls, (0, 8 - NUM_CLASSES)).reshape(1, 8)
    zt, o8 = _tc_post(acc_part, deg_part, zsem, W_pt, Wcls8,
                      b_gcn.reshape(1, HID), b_pt.reshape(1, ALIGN), bcls8)

    logits = o8[:N, :NUM_CLASSES]
    anomaly = o8[:N, NUM_CLASSES]
    return (logits, anomaly, zt[:N], zsem[:N])


# speedup vs baseline: 23.8249x; 1.0674x over previous
import functools

import jax
import jax.numpy as jnp
from jax import lax
from jax.experimental import pallas as pl
from jax.experimental.pallas import tpu as pltpu
from jax.experimental.pallas import tpu_sc as plsc

N = 10000
E = 320000
IN_DIM = 128
HID = 64
ALIGN = 32
NUM_CLASSES = 7

NC = 2
NS = 16
NW = NC * NS

NPAD = 10240
ROWS_PER_TILE = NPAD // NS
CH = 128
CPW = 80
EPW = CPW * CH
EPAD = NW * EPW
GARBAGE = N + 64
NB = 4

BR = 1024
GRID = NPAD // BR


def _sc_mesh():
    return plsc.VectorSubcoreMesh(core_axis_name="c", subcore_axis_name="s")


def _deg_body(dst_hbm, zeros_hbm, out_hbm, idx_v, ones_v, acc_sh, isem, asem):
    c = lax.axis_index("c")
    s = lax.axis_index("s")
    w = s * NC + c
    rslice = pl.ds(s * ROWS_PER_TILE, ROWS_PER_TILE)
    for i in range(8):
        ones_v[pl.ds(i * 16, 16)] = jnp.ones((16,), jnp.float32)
    pltpu.async_copy(zeros_hbm.at[rslice], acc_sh.at[rslice], isem).wait()
    pltpu.sync_copy(dst_hbm.at[pl.ds(w * CPW, CPW)], idx_v)
    plsc.subcore_barrier()

    @pl.loop(0, CPW, step=NB)
    def _chunks(t):
        hs = [
            pltpu.async_copy(ones_v, acc_sh.at[idx_v.at[t + b]], asem, add=True)
            for b in range(NB)
        ]
        for h in hs:
            h.wait()

    plsc.subcore_barrier()
    pltpu.sync_copy(acc_sh.at[rslice], out_hbm.at[c].at[rslice])


def _sc_degree(dst2d, zeros1d):
    return pl.kernel(
        _deg_body,
        out_type=jax.ShapeDtypeStruct((NC, NPAD), jnp.float32),
        mesh=_sc_mesh(),
        scratch_types=[
            pltpu.VMEM((CPW, CH), jnp.int32),
            pltpu.VMEM((CH,), jnp.float32),
            pltpu.VMEM_SHARED((NPAD,), jnp.float32),
            pltpu.SemaphoreType.DMA,
            pltpu.SemaphoreType.DMA,
        ],
    )(dst2d, zeros1d)


def _scat_body(y_hbm, src_hbm, dst_hbm, zeros_hbm, out_hbm,
               src_v, dst_v, rows_v, acc_sh, isem, gsem0, gsem1, ssem0, ssem1):
    c = lax.axis_index("c")
    s = lax.axis_index("s")
    w = s * NC + c
    rslice = pl.ds(s * ROWS_PER_TILE, ROWS_PER_TILE)
    gsems = (gsem0, gsem1)
    ssems = (ssem0, ssem1)
    NG = CPW // NB

    def fire_g(g, par):
        for b in range(NB):
            pltpu.async_copy(y_hbm.at[src_v.at[NB * g + b]],
                             rows_v.at[par * NB + b], gsems[par])

    def drain_g(g, par):
        for b in range(NB):
            pltpu.make_async_copy(y_hbm.at[src_v.at[NB * g + b]],
                                  rows_v.at[par * NB + b], gsems[par]).wait()

    def fire_s(g, par):
        for b in range(NB):
            pltpu.async_copy(rows_v.at[par * NB + b],
                             acc_sh.at[dst_v.at[NB * g + b]], ssems[par],
                             add=True)

    def drain_s(g, par):
        for b in range(NB):
            pltpu.make_async_copy(rows_v.at[par * NB + b],
                                  acc_sh.at[dst_v.at[NB * g + b]],
                                  ssems[par]).wait()

    @pl.when(c == 0)
    def _():
        pltpu.async_copy(y_hbm.at[rslice], acc_sh.at[rslice], isem).wait()

    @pl.when(c != 0)
    def _():
        pltpu.async_copy(zeros_hbm.at[rslice], acc_sh.at[rslice], isem).wait()

    pltpu.sync_copy(src_hbm.at[pl.ds(w * CPW, CPW)], src_v)
    pltpu.sync_copy(dst_hbm.at[pl.ds(w * CPW, CPW)], dst_v)
    plsc.subcore_barrier()

    fire_g(0, 0)
    fire_g(1, 1)
    drain_g(0, 0)
    fire_s(0, 0)

    @pl.loop(2, NG, step=2)
    def _groups(g):
        drain_s(g - 2, 0)
        fire_g(g, 0)
        drain_g(g - 1, 1)
        fire_s(g - 1, 1)
        drain_s(g - 1, 1)
        fire_g(g + 1, 1)
        drain_g(g, 0)
        fire_s(g, 0)

    drain_s(NG - 2, 0)
    drain_g(NG - 1, 1)
    fire_s(NG - 1, 1)
    drain_s(NG - 1, 1)

    plsc.subcore_barrier()
    pltpu.sync_copy(acc_sh.at[rslice], out_hbm.at[c].at[rslice])


def _sc_scatter(y, src2d, dst2d, zeros2d):
    return pl.kernel(
        _scat_body,
        out_type=jax.ShapeDtypeStruct((NC, NPAD, HID), jnp.float32),
        mesh=_sc_mesh(),
        compiler_params=pltpu.CompilerParams(use_tc_tiling_on_sc=False),
        scratch_types=[
            pltpu.VMEM((CPW, CH), jnp.int32),
            pltpu.VMEM((CPW, CH), jnp.int32),
            pltpu.VMEM((2 * NB, CH, HID), jnp.float32),
            pltpu.VMEM_SHARED((NPAD, HID), jnp.float32),
            pltpu.SemaphoreType.DMA,
            pltpu.SemaphoreType.DMA,
            pltpu.SemaphoreType.DMA,
            pltpu.SemaphoreType.DMA,
            pltpu.SemaphoreType.DMA,
        ],
    )(y, src2d, dst2d, zeros2d)


def _pre_body(x_ref, degp_ref, wg_ref, wps_ref, bps_ref, y_ref, zsem_ref):
    deg = degp_ref[0] + degp_ref[1] + 1.0
    dinv = lax.rsqrt(deg)
    xb = x_ref[...]
    xw = jnp.dot(xb, wg_ref[...], preferred_element_type=jnp.float32)
    y_ref[...] = xw * dinv[:, None]
    zsem_ref[...] = (
        jnp.dot(xb, wps_ref[...], preferred_element_type=jnp.float32)
        + bps_ref[...]
    )


def _tc_pre(xp, deg_part, W_gcn, W_ps, b_ps2):
    return pl.pallas_call(
        _pre_body,
        grid=(GRID,),
        in_specs=[
            pl.BlockSpec((BR, IN_DIM), lambda i: (i, 0)),
            pl.BlockSpec((NC, BR), lambda i: (0, i)),
            pl.BlockSpec((IN_DIM, HID), lambda i: (0, 0)),
            pl.BlockSpec((IN_DIM, ALIGN), lambda i: (0, 0)),
            pl.BlockSpec((1, ALIGN), lambda i: (0, 0)),
        ],
        out_specs=[
            pl.BlockSpec((BR, HID), lambda i: (i, 0)),
            pl.BlockSpec((BR, ALIGN), lambda i: (i, 0)),
        ],
        out_shape=[
            jax.ShapeDtypeStruct((NPAD, HID), jnp.float32),
            jax.ShapeDtypeStruct((NPAD, ALIGN), jnp.float32),
        ],
    )(xp, deg_part, W_gcn, W_ps, b_ps2)


def _post_body(acc_ref, degp_ref, zsem_ref, wpt_ref, wcls_ref,
               bg_ref, bpt_ref, bcls_ref, zt_ref, o8_ref):
    deg = degp_ref[0] + degp_ref[1] + 1.0
    dinv = lax.rsqrt(deg)
    agg = (acc_ref[0] + acc_ref[1]) * dinv[:, None] + bg_ref[...]
    h = jnp.maximum(agg, 0.0)
    zt = jnp.dot(h, wpt_ref[...], preferred_element_type=jnp.float32) + bpt_ref[...]
    zt_ref[...] = zt
    o8 = jnp.dot(zt, wcls_ref[...], preferred_element_type=jnp.float32) + bcls_ref[...]
    diff = zt - zsem_ref[...]
    an = jnp.sqrt(jnp.sum(diff * diff, axis=-1))
    col = lax.broadcasted_iota(jnp.int32, (BR, 8), 1)
    o8_ref[...] = jnp.where(col == NUM_CLASSES, an[:, None], o8)


def _tc_post(acc_part, deg_part, zsem, W_pt, Wcls8, b_gcn2, b_pt2, bcls8):
    return pl.pallas_call(
        _post_body,
        grid=(GRID,),
        in_specs=[
            pl.BlockSpec((NC, BR, HID), lambda i: (0, i, 0)),
            pl.BlockSpec((NC, BR), lambda i: (0, i)),
            pl.BlockSpec((BR, ALIGN), lambda i: (i, 0)),
            pl.BlockSpec((HID, ALIGN), lambda i: (0, 0)),
            pl.BlockSpec((ALIGN, 8), lambda i: (0, 0)),
            pl.BlockSpec((1, HID), lambda i: (0, 0)),
            pl.BlockSpec((1, ALIGN), lambda i: (0, 0)),
            pl.BlockSpec((1, 8), lambda i: (0, 0)),
        ],
        out_specs=[
            pl.BlockSpec((BR, ALIGN), lambda i: (i, 0)),
            pl.BlockSpec((BR, 8), lambda i: (i, 0)),
        ],
        out_shape=[
            jax.ShapeDtypeStruct((NPAD, ALIGN), jnp.float32),
            jax.ShapeDtypeStruct((NPAD, 8), jnp.float32),
        ],
    )(acc_part, deg_part, zsem, W_pt, Wcls8, b_gcn2, b_pt2, bcls8)


def kernel(x, edge_index, W_gcn, b_gcn, W_pt, b_pt, W_ps, b_ps, W_cls, b_cls):
    f32 = jnp.float32
    pad_e = EPAD - E
    src2d = jnp.concatenate(
        [edge_index[0], jnp.zeros((pad_e,), jnp.int32)]).reshape(EPAD // CH, CH)
    dst2d = jnp.concatenate(
        [edge_index[1], jnp.full((pad_e,), GARBAGE, jnp.int32)]
    ).reshape(EPAD // CH, CH)
    xp = jnp.pad(x, ((0, NPAD - N), (0, 0)))

    deg_part = _sc_degree(dst2d, jnp.zeros((NPAD,), f32))

    y, zsem = _tc_pre(xp, deg_part, W_gcn, W_ps, b_ps.reshape(1, ALIGN))

    acc_part = _sc_scatter(y, src2d, dst2d, jnp.zeros((NPAD, HID), f32))

    Wcls8 = jnp.pad(W_cls, ((0, 0), (0, 8 - NUM_CLASSES)))
    bcls8 = jnp.pad(b_cls, (0, 8 - NUM_CLASSES)).reshape(1, 8)
    zt, o8 = _tc_post(acc_part, deg_part, zsem, W_pt, Wcls8,
                      b_gcn.reshape(1, HID), b_pt.reshape(1, ALIGN), bcls8)

    logits = o8[:N, :NUM_CLASSES]
    anomaly = o8[:N, NUM_CLASSES]
    return (logits, anomaly, zt[:N], zsem[:N])


# speedup vs baseline: 48.4287x; 2.0327x over previous
import functools

import jax
import jax.numpy as jnp
from jax import lax
from jax.experimental import pallas as pl
from jax.experimental.pallas import tpu as pltpu
from jax.experimental.pallas import tpu_sc as plsc

N = 10000
E = 320000
IN_DIM = 128
HID = 64
ALIGN = 32
NUM_CLASSES = 7

NC = 2
NS = 16
NW = NC * NS

NPAD = 10240
ROWS_PER_TILE = NPAD // NS
CH = 125
CPW = 80
NB = 4

BR = 1024
GRID = NPAD // BR


def _sc_mesh():
    return plsc.VectorSubcoreMesh(core_axis_name="c", subcore_axis_name="s")


def _deg_body(dst_hbm, zeros_hbm, out_hbm, idx_v, ones_v, acc_sh, isem, asem):
    c = lax.axis_index("c")
    s = lax.axis_index("s")
    w = s * NC + c
    rslice = pl.ds(s * ROWS_PER_TILE, ROWS_PER_TILE)
    for i in range(8):
        ones_v[pl.ds(i * 16, 16)] = jnp.ones((16,), jnp.float32)
    pltpu.async_copy(zeros_hbm.at[rslice], acc_sh.at[rslice], isem).wait()
    pltpu.sync_copy(dst_hbm.at[pl.ds(w * CPW, CPW)], idx_v)
    plsc.subcore_barrier()

    @pl.loop(0, CPW, step=NB)
    def _chunks(t):
        hs = [
            pltpu.async_copy(ones_v.at[pl.ds(0, CH)], acc_sh.at[idx_v.at[t + b]],
                             asem, add=True)
            for b in range(NB)
        ]
        for h in hs:
            h.wait()

    plsc.subcore_barrier()
    pltpu.sync_copy(acc_sh.at[rslice], out_hbm.at[c].at[rslice])


def _sc_degree(dst2d, zeros1d):
    return pl.kernel(
        _deg_body,
        out_type=jax.ShapeDtypeStruct((NC, NPAD), jnp.float32),
        mesh=_sc_mesh(),
        compiler_params=pltpu.CompilerParams(use_tc_tiling_on_sc=False),
        scratch_types=[
            pltpu.VMEM((CPW, CH), jnp.int32),
            pltpu.VMEM((128,), jnp.float32),
            pltpu.VMEM_SHARED((NPAD,), jnp.float32),
            pltpu.SemaphoreType.DMA,
            pltpu.SemaphoreType.DMA,
        ],
    )(dst2d, zeros1d)


def _scat_body(y_hbm, src_hbm, dst_hbm, zeros_hbm, out_hbm,
               src_v, dst_v, rows_v, acc_sh, isem, gsem0, gsem1, ssem0, ssem1):
    c = lax.axis_index("c")
    s = lax.axis_index("s")
    w = s * NC + c
    rslice = pl.ds(s * ROWS_PER_TILE, ROWS_PER_TILE)
    gsems = (gsem0, gsem1)
    ssems = (ssem0, ssem1)
    NG = CPW // NB

    def fire_g(g, par):
        for b in range(NB):
            pltpu.async_copy(y_hbm.at[src_v.at[NB * g + b]],
                             rows_v.at[par * NB + b], gsems[par])

    def drain_g(g, par):
        for b in range(NB):
            pltpu.make_async_copy(y_hbm.at[src_v.at[NB * g + b]],
                                  rows_v.at[par * NB + b], gsems[par]).wait()

    def fire_s(g, par):
        for b in range(NB):
            pltpu.async_copy(rows_v.at[par * NB + b],
                             acc_sh.at[dst_v.at[NB * g + b]], ssems[par],
                             add=True)

    def drain_s(g, par):
        for b in range(NB):
            pltpu.make_async_copy(rows_v.at[par * NB + b],
                                  acc_sh.at[dst_v.at[NB * g + b]],
                                  ssems[par]).wait()

    @pl.when(c == 0)
    def _():
        pltpu.async_copy(y_hbm.at[rslice], acc_sh.at[rslice], isem).wait()

    @pl.when(c != 0)
    def _():
        pltpu.async_copy(zeros_hbm.at[rslice], acc_sh.at[rslice], isem).wait()

    pltpu.sync_copy(src_hbm.at[pl.ds(w * CPW, CPW)], src_v)
    pltpu.sync_copy(dst_hbm.at[pl.ds(w * CPW, CPW)], dst_v)
    plsc.subcore_barrier()

    fire_g(0, 0)
    fire_g(1, 1)
    drain_g(0, 0)
    fire_s(0, 0)

    @pl.loop(2, NG, step=2)
    def _groups(g):
        drain_s(g - 2, 0)
        fire_g(g, 0)
        drain_g(g - 1, 1)
        fire_s(g - 1, 1)
        drain_s(g - 1, 1)
        fire_g(g + 1, 1)
        drain_g(g, 0)
        fire_s(g, 0)

    drain_s(NG - 2, 0)
    drain_g(NG - 1, 1)
    fire_s(NG - 1, 1)
    drain_s(NG - 1, 1)

    plsc.subcore_barrier()
    pltpu.sync_copy(acc_sh.at[rslice], out_hbm.at[c].at[rslice])


def _sc_scatter(y, src2d, dst2d, zeros2d):
    return pl.kernel(
        _scat_body,
        out_type=jax.ShapeDtypeStruct((NC, NPAD, HID), jnp.float32),
        mesh=_sc_mesh(),
        compiler_params=pltpu.CompilerParams(use_tc_tiling_on_sc=False),
        scratch_types=[
            pltpu.VMEM((CPW, CH), jnp.int32),
            pltpu.VMEM((CPW, CH), jnp.int32),
            pltpu.VMEM((2 * NB, CH, HID), jnp.float32),
            pltpu.VMEM_SHARED((NPAD, HID), jnp.float32),
            pltpu.SemaphoreType.DMA,
            pltpu.SemaphoreType.DMA,
            pltpu.SemaphoreType.DMA,
            pltpu.SemaphoreType.DMA,
            pltpu.SemaphoreType.DMA,
        ],
    )(y, src2d, dst2d, zeros2d)


def _pre_body(x_ref, degp_ref, wg_ref, wps_ref, bps_ref, y_ref, zsem_ref):
    deg = degp_ref[0] + degp_ref[1] + 1.0
    dinv = lax.rsqrt(deg)
    xb = x_ref[...]
    xw = jnp.dot(xb, wg_ref[...], preferred_element_type=jnp.float32)
    y_ref[...] = xw * dinv[:, None]
    zsem_ref[...] = (
        jnp.dot(xb, wps_ref[...], preferred_element_type=jnp.float32)
        + bps_ref[...]
    )


def _tc_pre(xp, deg_part, W_gcn, W_ps, b_ps2):
    return pl.pallas_call(
        _pre_body,
        grid=(GRID,),
        in_specs=[
            pl.BlockSpec((BR, IN_DIM), lambda i: (i, 0)),
            pl.BlockSpec((NC, BR), lambda i: (0, i)),
            pl.BlockSpec((IN_DIM, HID), lambda i: (0, 0)),
            pl.BlockSpec((IN_DIM, ALIGN), lambda i: (0, 0)),
            pl.BlockSpec((1, ALIGN), lambda i: (0, 0)),
        ],
        out_specs=[
            pl.BlockSpec((BR, HID), lambda i: (i, 0)),
            pl.BlockSpec((BR, ALIGN), lambda i: (i, 0)),
        ],
        out_shape=[
            jax.ShapeDtypeStruct((NPAD, HID), jnp.float32),
            jax.ShapeDtypeStruct((NPAD, ALIGN), jnp.float32),
        ],
    )(xp, deg_part, W_gcn, W_ps, b_ps2)


def _post_body(acc_ref, degp_ref, zsem_ref, wpt_ref, wcls_ref,
               bg_ref, bpt_ref, bcls_ref, zt_ref, o8_ref):
    deg = degp_ref[0] + degp_ref[1] + 1.0
    dinv = lax.rsqrt(deg)
    agg = (acc_ref[0] + acc_ref[1]) * dinv[:, None] + bg_ref[...]
    h = jnp.maximum(agg, 0.0)
    zt = jnp.dot(h, wpt_ref[...], preferred_element_type=jnp.float32) + bpt_ref[...]
    zt_ref[...] = zt
    o8 = jnp.dot(zt, wcls_ref[...], preferred_element_type=jnp.float32) + bcls_ref[...]
    diff = zt - zsem_ref[...]
    an = jnp.sqrt(jnp.sum(diff * diff, axis=-1))
    col = lax.broadcasted_iota(jnp.int32, (BR, 8), 1)
    o8_ref[...] = jnp.where(col == NUM_CLASSES, an[:, None], o8)


def _tc_post(acc_part, deg_part, zsem, W_pt, Wcls8, b_gcn2, b_pt2, bcls8):
    return pl.pallas_call(
        _post_body,
        grid=(GRID,),
        in_specs=[
            pl.BlockSpec((NC, BR, HID), lambda i: (0, i, 0)),
            pl.BlockSpec((NC, BR), lambda i: (0, i)),
            pl.BlockSpec((BR, ALIGN), lambda i: (i, 0)),
            pl.BlockSpec((HID, ALIGN), lambda i: (0, 0)),
            pl.BlockSpec((ALIGN, 8), lambda i: (0, 0)),
            pl.BlockSpec((1, HID), lambda i: (0, 0)),
            pl.BlockSpec((1, ALIGN), lambda i: (0, 0)),
            pl.BlockSpec((1, 8), lambda i: (0, 0)),
        ],
        out_specs=[
            pl.BlockSpec((BR, ALIGN), lambda i: (i, 0)),
            pl.BlockSpec((BR, 8), lambda i: (i, 0)),
        ],
        out_shape=[
            jax.ShapeDtypeStruct((NPAD, ALIGN), jnp.float32),
            jax.ShapeDtypeStruct((NPAD, 8), jnp.float32),
        ],
    )(acc_part, deg_part, zsem, W_pt, Wcls8, b_gcn2, b_pt2, bcls8)


def kernel(x, edge_index, W_gcn, b_gcn, W_pt, b_pt, W_ps, b_ps, W_cls, b_cls):
    f32 = jnp.float32
    src2d = edge_index[0].reshape(E // CH, CH)
    dst2d = edge_index[1].reshape(E // CH, CH)
    xp = jnp.pad(x, ((0, NPAD - N), (0, 0)))

    deg_part = _sc_degree(dst2d, jnp.zeros((NPAD,), f32))

    y, zsem = _tc_pre(xp, deg_part, W_gcn, W_ps, b_ps.reshape(1, ALIGN))

    acc_part = _sc_scatter(y, src2d, dst2d, jnp.zeros((NPAD, HID), f32))

    Wcls8 = jnp.pad(W_cls, ((0, 0), (0, 8 - NUM_CLASSES)))
    bcls8 = jnp.pad(b_cls, (0, 8 - NUM_CLASSES)).reshape(1, 8)
    zt, o8 = _tc_post(acc_part, deg_part, zsem, W_pt, Wcls8,
                      b_gcn.reshape(1, HID), b_pt.reshape(1, ALIGN), bcls8)

    logits = o8[:N, :NUM_CLASSES]
    anomaly = o8[:N, NUM_CLASSES]
    return (logits, anomaly, zt[:N], zsem[:N])


# speedup vs baseline: 52.0460x; 1.0747x over previous
import functools

import jax
import jax.numpy as jnp
from jax import lax
from jax.experimental import pallas as pl
from jax.experimental.pallas import tpu as pltpu
from jax.experimental.pallas import tpu_sc as plsc

N = 10000
E = 320000
IN_DIM = 128
HID = 64
ALIGN = 32
NUM_CLASSES = 7

NC = 2
NS = 16
NW = NC * NS

DEGPAD = 10240
DROWS = DEGPAD // NS
RPT = N // NS
CH = 125
CPW = 80
NB = 4

BR = 1024
GRID = (N + BR - 1) // BR


def _sc_mesh():
    return plsc.VectorSubcoreMesh(core_axis_name="c", subcore_axis_name="s")


def _deg_body(dst_hbm, zeros_hbm, out_hbm, idx_v, ones_v, acc_sh, isem, asem):
    c = lax.axis_index("c")
    s = lax.axis_index("s")
    w = s * NC + c
    rslice = pl.ds(s * DROWS, DROWS)
    for i in range(8):
        ones_v[pl.ds(i * 16, 16)] = jnp.ones((16,), jnp.float32)
    pltpu.async_copy(zeros_hbm.at[rslice], acc_sh.at[rslice], isem).wait()
    pltpu.sync_copy(dst_hbm.at[pl.ds(w * CPW, CPW)], idx_v)
    plsc.subcore_barrier()

    @pl.loop(0, CPW, step=NB)
    def _chunks(t):
        hs = [
            pltpu.async_copy(ones_v.at[pl.ds(0, CH)], acc_sh.at[idx_v.at[t + b]],
                             asem, add=True)
            for b in range(NB)
        ]
        for h in hs:
            h.wait()

    plsc.subcore_barrier()
    pltpu.sync_copy(acc_sh.at[rslice], out_hbm.at[c].at[rslice])


def _sc_degree(dst2d, zeros1d):
    return pl.kernel(
        _deg_body,
        out_type=jax.ShapeDtypeStruct((NC, DEGPAD), jnp.float32),
        mesh=_sc_mesh(),
        compiler_params=pltpu.CompilerParams(use_tc_tiling_on_sc=False),
        scratch_types=[
            pltpu.VMEM((CPW, CH), jnp.int32),
            pltpu.VMEM((128,), jnp.float32),
            pltpu.VMEM_SHARED((DEGPAD,), jnp.float32),
            pltpu.SemaphoreType.DMA,
            pltpu.SemaphoreType.DMA,
        ],
    )(dst2d, zeros1d)


def _scat_body(y_hbm, src_hbm, dst_hbm, zeros_hbm, out_hbm,
               src_v, dst_v, rows_v, acc_sh, isem, gsem0, gsem1, ssem0, ssem1):
    c = lax.axis_index("c")
    s = lax.axis_index("s")
    w = s * NC + c
    rslice = pl.ds(s * RPT, RPT)
    gsems = (gsem0, gsem1)
    ssems = (ssem0, ssem1)
    NG = CPW // NB

    def fire_g(g, par):
        for b in range(NB):
            pltpu.async_copy(y_hbm.at[src_v.at[NB * g + b]],
                             rows_v.at[par * NB + b], gsems[par])

    def drain_g(g, par):
        for b in range(NB):
            pltpu.make_async_copy(y_hbm.at[src_v.at[NB * g + b]],
                                  rows_v.at[par * NB + b], gsems[par]).wait()

    def fire_s(g, par):
        for b in range(NB):
            pltpu.async_copy(rows_v.at[par * NB + b],
                             acc_sh.at[dst_v.at[NB * g + b]], ssems[par],
                             add=True)

    def drain_s(g, par):
        for b in range(NB):
            pltpu.make_async_copy(rows_v.at[par * NB + b],
                                  acc_sh.at[dst_v.at[NB * g + b]],
                                  ssems[par]).wait()

    @pl.when(c == 0)
    def _():
        pltpu.async_copy(y_hbm.at[rslice], acc_sh.at[rslice], isem).wait()

    @pl.when(c != 0)
    def _():
        pltpu.async_copy(zeros_hbm.at[rslice], acc_sh.at[rslice], isem).wait()

    pltpu.sync_copy(src_hbm.at[pl.ds(w * CPW, CPW)], src_v)
    pltpu.sync_copy(dst_hbm.at[pl.ds(w * CPW, CPW)], dst_v)
    plsc.subcore_barrier()

    fire_g(0, 0)
    fire_g(1, 1)
    drain_g(0, 0)
    fire_s(0, 0)

    @pl.loop(2, NG, step=2)
    def _groups(g):
        drain_s(g - 2, 0)
        fire_g(g, 0)
        drain_g(g - 1, 1)
        fire_s(g - 1, 1)
        drain_s(g - 1, 1)
        fire_g(g + 1, 1)
        drain_g(g, 0)
        fire_s(g, 0)

    drain_s(NG - 2, 0)
    drain_g(NG - 1, 1)
    fire_s(NG - 1, 1)
    drain_s(NG - 1, 1)

    plsc.subcore_barrier()
    pltpu.sync_copy(acc_sh.at[rslice], out_hbm.at[c].at[rslice])


def _sc_scatter(y, src2d, dst2d, zeros2d):
    return pl.kernel(
        _scat_body,
        out_type=jax.ShapeDtypeStruct((NC, N, HID), jnp.float32),
        mesh=_sc_mesh(),
        compiler_params=pltpu.CompilerParams(use_tc_tiling_on_sc=False),
        scratch_types=[
            pltpu.VMEM((CPW, CH), jnp.int32),
            pltpu.VMEM((CPW, CH), jnp.int32),
            pltpu.VMEM((2 * NB, CH, HID), jnp.float32),
            pltpu.VMEM_SHARED((N, HID), jnp.float32),
            pltpu.SemaphoreType.DMA,
            pltpu.SemaphoreType.DMA,
            pltpu.SemaphoreType.DMA,
            pltpu.SemaphoreType.DMA,
            pltpu.SemaphoreType.DMA,
        ],
    )(y, src2d, dst2d, zeros2d)


def _pre_body(x_ref, degp_ref, wg_ref, wps_ref, bps_ref, y_ref, zsem_ref):
    deg = degp_ref[0] + degp_ref[1] + 1.0
    dinv = lax.rsqrt(deg)
    xb = x_ref[...]
    xw = jnp.dot(xb, wg_ref[...], preferred_element_type=jnp.float32)
    y_ref[...] = xw * dinv[:, None]
    zsem_ref[...] = (
        jnp.dot(xb, wps_ref[...], preferred_element_type=jnp.float32)
        + bps_ref[...]
    )


def _tc_pre(xp, deg_part, W_gcn, W_ps, b_ps2):
    return pl.pallas_call(
        _pre_body,
        grid=(GRID,),
        in_specs=[
            pl.BlockSpec((BR, IN_DIM), lambda i: (i, 0)),
            pl.BlockSpec((NC, BR), lambda i: (0, i)),
            pl.BlockSpec((IN_DIM, HID), lambda i: (0, 0)),
            pl.BlockSpec((IN_DIM, ALIGN), lambda i: (0, 0)),
            pl.BlockSpec((1, ALIGN), lambda i: (0, 0)),
        ],
        out_specs=[
            pl.BlockSpec((BR, HID), lambda i: (i, 0)),
            pl.BlockSpec((BR, ALIGN), lambda i: (i, 0)),
        ],
        out_shape=[
            jax.ShapeDtypeStruct((N, HID), jnp.float32),
            jax.ShapeDtypeStruct((N, ALIGN), jnp.float32),
        ],
    )(xp, deg_part, W_gcn, W_ps, b_ps2)


def _post_body(acc_ref, degp_ref, zsem_ref, wpt_ref, wcls_ref,
               bg_ref, bpt_ref, bcls_ref, zt_ref, lg_ref, an_ref):
    deg = degp_ref[0] + degp_ref[1] + 1.0
    dinv = lax.rsqrt(deg)
    agg = (acc_ref[0] + acc_ref[1]) * dinv[:, None] + bg_ref[...]
    h = jnp.maximum(agg, 0.0)
    zt = jnp.dot(h, wpt_ref[...], preferred_element_type=jnp.float32) + bpt_ref[...]
    zt_ref[...] = zt
    lg_ref[...] = (
        jnp.dot(zt, wcls_ref[...], preferred_element_type=jnp.float32)
        + bcls_ref[...]
    )
    diff = zt - zsem_ref[...]
    an_ref[...] = jnp.sqrt(jnp.sum(diff * diff, axis=-1))


def _tc_post(acc_part, deg_part, zsem, W_pt, W_cls, b_gcn2, b_pt2, bcls2):
    return pl.pallas_call(
        _post_body,
        grid=(GRID,),
        in_specs=[
            pl.BlockSpec((NC, BR, HID), lambda i: (0, i, 0)),
            pl.BlockSpec((NC, BR), lambda i: (0, i)),
            pl.BlockSpec((BR, ALIGN), lambda i: (i, 0)),
            pl.BlockSpec((HID, ALIGN), lambda i: (0, 0)),
            pl.BlockSpec((ALIGN, NUM_CLASSES), lambda i: (0, 0)),
            pl.BlockSpec((1, HID), lambda i: (0, 0)),
            pl.BlockSpec((1, ALIGN), lambda i: (0, 0)),
            pl.BlockSpec((1, NUM_CLASSES), lambda i: (0, 0)),
        ],
        out_specs=[
            pl.BlockSpec((BR, ALIGN), lambda i: (i, 0)),
            pl.BlockSpec((BR, NUM_CLASSES), lambda i: (i, 0)),
            pl.BlockSpec((BR,), lambda i: (i,)),
        ],
        out_shape=[
            jax.ShapeDtypeStruct((N, ALIGN), jnp.float32),
            jax.ShapeDtypeStruct((N, NUM_CLASSES), jnp.float32),
            jax.ShapeDtypeStruct((N,), jnp.float32),
        ],
    )(acc_part, deg_part, zsem, W_pt, W_cls, b_gcn2, b_pt2, bcls2)


def kernel(x, edge_index, W_gcn, b_gcn, W_pt, b_pt, W_ps, b_ps, W_cls, b_cls):
    f32 = jnp.float32
    src2d = edge_index[0].reshape(E // CH, CH)
    dst2d = edge_index[1].reshape(E // CH, CH)

    deg_part = _sc_degree(dst2d, jnp.zeros((DEGPAD,), f32))

    y, zsem = _tc_pre(x, deg_part, W_gcn, W_ps, b_ps.reshape(1, ALIGN))

    acc_part = _sc_scatter(y, src2d, dst2d, jnp.zeros((N, HID), f32))

    zt, logits, anomaly = _tc_post(
        acc_part, deg_part, zsem, W_pt, W_cls,
        b_gcn.reshape(1, HID), b_pt.reshape(1, ALIGN),
        b_cls.reshape(1, NUM_CLASSES))

    return (logits, anomaly, zt, zsem)


# speedup vs baseline: 55.3255x; 1.0630x over previous
import functools

import jax
import jax.numpy as jnp
from jax import lax
from jax.experimental import pallas as pl
from jax.experimental.pallas import tpu as pltpu
from jax.experimental.pallas import tpu_sc as plsc

N = 10000
E = 320000
IN_DIM = 128
HID = 64
ALIGN = 32
NUM_CLASSES = 7

NC = 2
NS = 16
NW = NC * NS

DEGPAD = 10240
DROWS = DEGPAD // NS
RPT = N // NS
CH = 125
CPW = 80
NB = 4

BR = 1024
GRID = (N + BR - 1) // BR


def _sc_mesh():
    return plsc.VectorSubcoreMesh(core_axis_name="c", subcore_axis_name="s")


def _deg_body(dst_hbm, zeros_hbm, out0_hbm, out1_hbm, idx_v, ones_v, acc_sh, isem, asem):
    c = lax.axis_index("c")
    s = lax.axis_index("s")
    w = s * NC + c
    rslice = pl.ds(s * DROWS, DROWS)
    for i in range(8):
        ones_v[pl.ds(i * 16, 16)] = jnp.ones((16,), jnp.float32)
    pltpu.async_copy(zeros_hbm.at[rslice], acc_sh.at[rslice], isem).wait()
    pltpu.sync_copy(dst_hbm.at[pl.ds(w * CPW, CPW)], idx_v)
    plsc.subcore_barrier()

    @pl.loop(0, CPW, step=NB)
    def _chunks(t):
        hs = [
            pltpu.async_copy(ones_v.at[pl.ds(0, CH)], acc_sh.at[idx_v.at[t + b]],
                             asem, add=True)
            for b in range(NB)
        ]
        for h in hs:
            h.wait()

    plsc.subcore_barrier()

    @pl.when(c == 0)
    def _():
        pltpu.sync_copy(acc_sh.at[rslice], out0_hbm.at[rslice])

    @pl.when(c != 0)
    def _():
        pltpu.sync_copy(acc_sh.at[rslice], out1_hbm.at[rslice])


def _sc_degree(dst2d, zeros1d):
    return pl.kernel(
        _deg_body,
        out_type=[jax.ShapeDtypeStruct((DEGPAD,), jnp.float32),
                  jax.ShapeDtypeStruct((DEGPAD,), jnp.float32)],
        mesh=_sc_mesh(),
        compiler_params=pltpu.CompilerParams(use_tc_tiling_on_sc=False),
        scratch_types=[
            pltpu.VMEM((CPW, CH), jnp.int32),
            pltpu.VMEM((128,), jnp.float32),
            pltpu.VMEM_SHARED((DEGPAD,), jnp.float32),
            pltpu.SemaphoreType.DMA,
            pltpu.SemaphoreType.DMA,
        ],
    )(dst2d, zeros1d)


def _scat_body(y_hbm, src_hbm, dst_hbm, zeros_hbm, out_hbm,
               src_v, dst_v, rows_v, acc_sh, isem, gsem0, gsem1, ssem0, ssem1):
    c = lax.axis_index("c")
    s = lax.axis_index("s")
    w = s * NC + c
    rslice = pl.ds(s * RPT, RPT)
    gsems = (gsem0, gsem1)
    ssems = (ssem0, ssem1)
    NG = CPW // NB

    def fire_g(g, par):
        for b in range(NB):
            pltpu.async_copy(y_hbm.at[src_v.at[NB * g + b]],
                             rows_v.at[par * NB + b], gsems[par])

    def drain_g(g, par):
        for b in range(NB):
            pltpu.make_async_copy(y_hbm.at[src_v.at[NB * g + b]],
                                  rows_v.at[par * NB + b], gsems[par]).wait()

    def fire_s(g, par):
        for b in range(NB):
            pltpu.async_copy(rows_v.at[par * NB + b],
                             acc_sh.at[dst_v.at[NB * g + b]], ssems[par],
                             add=True)

    def drain_s(g, par):
        for b in range(NB):
            pltpu.make_async_copy(rows_v.at[par * NB + b],
                                  acc_sh.at[dst_v.at[NB * g + b]],
                                  ssems[par]).wait()

    @pl.when(c == 0)
    def _():
        pltpu.async_copy(y_hbm.at[rslice], acc_sh.at[rslice], isem).wait()

    @pl.when(c != 0)
    def _():
        pltpu.async_copy(zeros_hbm.at[rslice], acc_sh.at[rslice], isem).wait()

    pltpu.sync_copy(src_hbm.at[pl.ds(w * CPW, CPW)], src_v)
    pltpu.sync_copy(dst_hbm.at[pl.ds(w * CPW, CPW)], dst_v)
    plsc.subcore_barrier()

    fire_g(0, 0)
    fire_g(1, 1)
    drain_g(0, 0)
    fire_s(0, 0)

    @pl.loop(2, NG, step=2)
    def _groups(g):
        drain_s(g - 2, 0)
        fire_g(g, 0)
        drain_g(g - 1, 1)
        fire_s(g - 1, 1)
        drain_s(g - 1, 1)
        fire_g(g + 1, 1)
        drain_g(g, 0)
        fire_s(g, 0)

    drain_s(NG - 2, 0)
    drain_g(NG - 1, 1)
    fire_s(NG - 1, 1)
    drain_s(NG - 1, 1)

    plsc.subcore_barrier()
    pltpu.sync_copy(acc_sh.at[rslice],
                    out_hbm.at[pl.ds(s * RPT, RPT), pl.ds(c * HID, HID)])


def _sc_scatter(y, src2d, dst2d, zeros2d):
    return pl.kernel(
        _scat_body,
        out_type=jax.ShapeDtypeStruct((N, 2 * HID), jnp.float32),
        mesh=_sc_mesh(),
        compiler_params=pltpu.CompilerParams(use_tc_tiling_on_sc=False),
        scratch_types=[
            pltpu.VMEM((CPW, CH), jnp.int32),
            pltpu.VMEM((CPW, CH), jnp.int32),
            pltpu.VMEM((2 * NB, CH, HID), jnp.float32),
            pltpu.VMEM_SHARED((N, HID), jnp.float32),
            pltpu.SemaphoreType.DMA,
            pltpu.SemaphoreType.DMA,
            pltpu.SemaphoreType.DMA,
            pltpu.SemaphoreType.DMA,
            pltpu.SemaphoreType.DMA,
        ],
    )(y, src2d, dst2d, zeros2d)


def _pre_body(x_ref, deg0_ref, deg1_ref, wg_ref, wps_ref, bps_ref,
              y_ref, zsem_ref):
    deg = deg0_ref[...] + deg1_ref[...] + 1.0
    dinv = lax.rsqrt(deg)
    xb = x_ref[...]
    xw = jnp.dot(xb, wg_ref[...], preferred_element_type=jnp.float32)
    y_ref[...] = xw * dinv[:, None]
    zsem_ref[...] = (
        jnp.dot(xb, wps_ref[...], preferred_element_type=jnp.float32)
        + bps_ref[...]
    )


def _tc_pre(xp, deg0, deg1, W_gcn, W_ps, b_ps2):
    return pl.pallas_call(
        _pre_body,
        grid=(GRID,),
        in_specs=[
            pl.BlockSpec((BR, IN_DIM), lambda i: (i, 0)),
            pl.BlockSpec((BR,), lambda i: (i,)),
            pl.BlockSpec((BR,), lambda i: (i,)),
            pl.BlockSpec((IN_DIM, HID), lambda i: (0, 0)),
            pl.BlockSpec((IN_DIM, ALIGN), lambda i: (0, 0)),
            pl.BlockSpec((1, ALIGN), lambda i: (0, 0)),
        ],
        out_specs=[
            pl.BlockSpec((BR, HID), lambda i: (i, 0)),
            pl.BlockSpec((BR, ALIGN), lambda i: (i, 0)),
        ],
        out_shape=[
            jax.ShapeDtypeStruct((N, HID), jnp.float32),
            jax.ShapeDtypeStruct((N, ALIGN), jnp.float32),
        ],
    )(xp, deg0, deg1, W_gcn, W_ps, b_ps2)


def _post_body(acc_ref, deg0_ref, deg1_ref, zsem_ref, wpt_ref, wcls_ref,
               bg_ref, bpt_ref, bcls_ref, zt_ref, lg_ref, an_ref):
    deg = deg0_ref[...] + deg1_ref[...] + 1.0
    dinv = lax.rsqrt(deg)
    a2 = acc_ref[...]
    agg = (a2[:, :HID] + a2[:, HID:]) * dinv[:, None] + bg_ref[...]
    h = jnp.maximum(agg, 0.0)
    zt = jnp.dot(h, wpt_ref[...], preferred_element_type=jnp.float32) + bpt_ref[...]
    zt_ref[...] = zt
    lg_ref[...] = (
        jnp.dot(zt, wcls_ref[...], preferred_element_type=jnp.float32)
        + bcls_ref[...]
    )
    diff = zt - zsem_ref[...]
    an_ref[...] = jnp.sqrt(jnp.sum(diff * diff, axis=-1))


def _tc_post(acc2, deg0, deg1, zsem, W_pt, W_cls, b_gcn2, b_pt2, bcls2):
    return pl.pallas_call(
        _post_body,
        grid=(GRID,),
        in_specs=[
            pl.BlockSpec((BR, 2 * HID), lambda i: (i, 0)),
            pl.BlockSpec((BR,), lambda i: (i,)),
            pl.BlockSpec((BR,), lambda i: (i,)),
            pl.BlockSpec((BR, ALIGN), lambda i: (i, 0)),
            pl.BlockSpec((HID, ALIGN), lambda i: (0, 0)),
            pl.BlockSpec((ALIGN, NUM_CLASSES), lambda i: (0, 0)),
            pl.BlockSpec((1, HID), lambda i: (0, 0)),
            pl.BlockSpec((1, ALIGN), lambda i: (0, 0)),
            pl.BlockSpec((1, NUM_CLASSES), lambda i: (0, 0)),
        ],
        out_specs=[
            pl.BlockSpec((BR, ALIGN), lambda i: (i, 0)),
            pl.BlockSpec((BR, NUM_CLASSES), lambda i: (i, 0)),
            pl.BlockSpec((BR,), lambda i: (i,)),
        ],
        out_shape=[
            jax.ShapeDtypeStruct((N, ALIGN), jnp.float32),
            jax.ShapeDtypeStruct((N, NUM_CLASSES), jnp.float32),
            jax.ShapeDtypeStruct((N,), jnp.float32),
        ],
    )(acc2, deg0, deg1, zsem, W_pt, W_cls, b_gcn2, b_pt2, bcls2)


def kernel(x, edge_index, W_gcn, b_gcn, W_pt, b_pt, W_ps, b_ps, W_cls, b_cls):
    f32 = jnp.float32
    src2d = edge_index[0].reshape(E // CH, CH)
    dst2d = edge_index[1].reshape(E // CH, CH)

    deg0, deg1 = _sc_degree(dst2d, jnp.zeros((DEGPAD,), f32))

    y, zsem = _tc_pre(x, deg0, deg1, W_gcn, W_ps, b_ps.reshape(1, ALIGN))

    acc2 = _sc_scatter(y, src2d, dst2d, jnp.zeros((N, HID), f32))

    zt, logits, anomaly = _tc_post(
        acc2, deg0, deg1, zsem, W_pt, W_cls,
        b_gcn.reshape(1, HID), b_pt.reshape(1, ALIGN),
        b_cls.reshape(1, NUM_CLASSES))

    return (logits, anomaly, zt, zsem)


# speedup vs baseline: 57.4371x; 1.0382x over previous
import functools

import jax
import jax.numpy as jnp
from jax import lax
from jax.experimental import pallas as pl
from jax.experimental.pallas import tpu as pltpu
from jax.experimental.pallas import tpu_sc as plsc

N = 10000
E = 320000
IN_DIM = 128
HID = 64
ALIGN = 32
NUM_CLASSES = 7

NC = 2
NS = 16
NW = NC * NS

DEGPAD = 10240
DROWS = DEGPAD // NS
RPT = N // NS
CH = 125
CPW = 80
NB = 4

BR = 2048
GRID = (N + BR - 1) // BR


def _sc_mesh():
    return plsc.VectorSubcoreMesh(core_axis_name="c", subcore_axis_name="s")


def _deg_body(dst_hbm, zeros_hbm, out0_hbm, out1_hbm, idx_v, ones_v, acc_sh, isem, asem):
    c = lax.axis_index("c")
    s = lax.axis_index("s")
    w = s * NC + c
    rslice = pl.ds(s * DROWS, DROWS)
    for i in range(8):
        ones_v[pl.ds(i * 16, 16)] = jnp.ones((16,), jnp.float32)
    pltpu.async_copy(zeros_hbm.at[rslice], acc_sh.at[rslice], isem).wait()
    pltpu.sync_copy(dst_hbm.at[pl.ds(w * CPW, CPW)], idx_v)
    plsc.subcore_barrier()

    @pl.loop(0, CPW, step=NB)
    def _chunks(t):
        hs = [
            pltpu.async_copy(ones_v.at[pl.ds(0, CH)], acc_sh.at[idx_v.at[t + b]],
                             asem, add=True)
            for b in range(NB)
        ]
        for h in hs:
            h.wait()

    plsc.subcore_barrier()

    @pl.when(c == 0)
    def _():
        pltpu.sync_copy(acc_sh.at[rslice], out0_hbm.at[rslice])

    @pl.when(c != 0)
    def _():
        pltpu.sync_copy(acc_sh.at[rslice], out1_hbm.at[rslice])


def _sc_degree(dst2d, zeros1d):
    return pl.kernel(
        _deg_body,
        out_type=[jax.ShapeDtypeStruct((DEGPAD,), jnp.float32),
                  jax.ShapeDtypeStruct((DEGPAD,), jnp.float32)],
        mesh=_sc_mesh(),
        compiler_params=pltpu.CompilerParams(use_tc_tiling_on_sc=False),
        scratch_types=[
            pltpu.VMEM((CPW, CH), jnp.int32),
            pltpu.VMEM((128,), jnp.float32),
            pltpu.VMEM_SHARED((DEGPAD,), jnp.float32),
            pltpu.SemaphoreType.DMA,
            pltpu.SemaphoreType.DMA,
        ],
    )(dst2d, zeros1d)


def _scat_body(y_hbm, src_hbm, dst_hbm, zeros_hbm, out_hbm,
               src_v, dst_v, rows_v, acc_sh, isem, gsem0, gsem1, ssem0, ssem1):
    c = lax.axis_index("c")
    s = lax.axis_index("s")
    w = s * NC + c
    rslice = pl.ds(s * RPT, RPT)
    gsems = (gsem0, gsem1)
    ssems = (ssem0, ssem1)
    NG = CPW // NB

    def fire_g(g, par):
        for b in range(NB):
            pltpu.async_copy(y_hbm.at[src_v.at[NB * g + b]],
                             rows_v.at[par * NB + b], gsems[par])

    def drain_g(g, par):
        for b in range(NB):
            pltpu.make_async_copy(y_hbm.at[src_v.at[NB * g + b]],
                                  rows_v.at[par * NB + b], gsems[par]).wait()

    def fire_s(g, par):
        for b in range(NB):
            pltpu.async_copy(rows_v.at[par * NB + b],
                             acc_sh.at[dst_v.at[NB * g + b]], ssems[par],
                             add=True)

    def drain_s(g, par):
        for b in range(NB):
            pltpu.make_async_copy(rows_v.at[par * NB + b],
                                  acc_sh.at[dst_v.at[NB * g + b]],
                                  ssems[par]).wait()

    @pl.when(c == 0)
    def _():
        pltpu.async_copy(y_hbm.at[rslice], acc_sh.at[rslice], isem).wait()

    @pl.when(c != 0)
    def _():
        pltpu.async_copy(zeros_hbm.at[rslice], acc_sh.at[rslice], isem).wait()

    pltpu.sync_copy(src_hbm.at[pl.ds(w * CPW, CPW)], src_v)
    pltpu.sync_copy(dst_hbm.at[pl.ds(w * CPW, CPW)], dst_v)
    plsc.subcore_barrier()

    fire_g(0, 0)
    fire_g(1, 1)
    drain_g(0, 0)
    fire_s(0, 0)

    @pl.loop(2, NG, step=2)
    def _groups(g):
        drain_s(g - 2, 0)
        fire_g(g, 0)
        drain_g(g - 1, 1)
        fire_s(g - 1, 1)
        drain_s(g - 1, 1)
        fire_g(g + 1, 1)
        drain_g(g, 0)
        fire_s(g, 0)

    drain_s(NG - 2, 0)
    drain_g(NG - 1, 1)
    fire_s(NG - 1, 1)
    drain_s(NG - 1, 1)

    plsc.subcore_barrier()
    pltpu.sync_copy(acc_sh.at[rslice],
                    out_hbm.at[pl.ds(s * RPT, RPT), pl.ds(c * HID, HID)])


def _sc_scatter(y, src2d, dst2d, zeros2d):
    return pl.kernel(
        _scat_body,
        out_type=jax.ShapeDtypeStruct((N, 2 * HID), jnp.float32),
        mesh=_sc_mesh(),
        compiler_params=pltpu.CompilerParams(use_tc_tiling_on_sc=False),
        scratch_types=[
            pltpu.VMEM((CPW, CH), jnp.int32),
            pltpu.VMEM((CPW, CH), jnp.int32),
            pltpu.VMEM((2 * NB, CH, HID), jnp.float32),
            pltpu.VMEM_SHARED((N, HID), jnp.float32),
            pltpu.SemaphoreType.DMA,
            pltpu.SemaphoreType.DMA,
            pltpu.SemaphoreType.DMA,
            pltpu.SemaphoreType.DMA,
            pltpu.SemaphoreType.DMA,
        ],
    )(y, src2d, dst2d, zeros2d)


def _pre_body(x_ref, deg0_ref, deg1_ref, wg_ref, wps_ref, bps_ref,
              y_ref, zsem_ref):
    deg = deg0_ref[...] + deg1_ref[...] + 1.0
    dinv = lax.rsqrt(deg)
    xb = x_ref[...]
    xw = jnp.dot(xb, wg_ref[...], preferred_element_type=jnp.float32)
    y_ref[...] = xw * dinv[:, None]
    zsem_ref[...] = (
        jnp.dot(xb, wps_ref[...], preferred_element_type=jnp.float32)
        + bps_ref[...]
    )


def _tc_pre(xp, deg0, deg1, W_gcn, W_ps, b_ps2):
    return pl.pallas_call(
        _pre_body,
        grid=(GRID,),
        in_specs=[
            pl.BlockSpec((BR, IN_DIM), lambda i: (i, 0)),
            pl.BlockSpec((BR,), lambda i: (i,)),
            pl.BlockSpec((BR,), lambda i: (i,)),
            pl.BlockSpec((IN_DIM, HID), lambda i: (0, 0)),
            pl.BlockSpec((IN_DIM, ALIGN), lambda i: (0, 0)),
            pl.BlockSpec((1, ALIGN), lambda i: (0, 0)),
        ],
        out_specs=[
            pl.BlockSpec((BR, HID), lambda i: (i, 0)),
            pl.BlockSpec((BR, ALIGN), lambda i: (i, 0)),
        ],
        out_shape=[
            jax.ShapeDtypeStruct((N, HID), jnp.float32),
            jax.ShapeDtypeStruct((N, ALIGN), jnp.float32),
        ],
    )(xp, deg0, deg1, W_gcn, W_ps, b_ps2)


def _post_body(acc_ref, deg0_ref, deg1_ref, zsem_ref, wpt_ref, wcls_ref,
               bg_ref, bpt_ref, bcls_ref, zt_ref, lg_ref, an_ref):
    deg = deg0_ref[...] + deg1_ref[...] + 1.0
    dinv = lax.rsqrt(deg)
    a2 = acc_ref[...]
    agg = (a2[:, :HID] + a2[:, HID:]) * dinv[:, None] + bg_ref[...]
    h = jnp.maximum(agg, 0.0)
    zt = jnp.dot(h, wpt_ref[...], preferred_element_type=jnp.float32) + bpt_ref[...]
    zt_ref[...] = zt
    lg_ref[...] = (
        jnp.dot(zt, wcls_ref[...], preferred_element_type=jnp.float32)
        + bcls_ref[...]
    )
    diff = zt - zsem_ref[...]
    an_ref[...] = jnp.sqrt(jnp.sum(diff * diff, axis=-1))


def _tc_post(acc2, deg0, deg1, zsem, W_pt, W_cls, b_gcn2, b_pt2, bcls2):
    return pl.pallas_call(
        _post_body,
        grid=(GRID,),
        in_specs=[
            pl.BlockSpec((BR, 2 * HID), lambda i: (i, 0)),
            pl.BlockSpec((BR,), lambda i: (i,)),
            pl.BlockSpec((BR,), lambda i: (i,)),
            pl.BlockSpec((BR, ALIGN), lambda i: (i, 0)),
            pl.BlockSpec((HID, ALIGN), lambda i: (0, 0)),
            pl.BlockSpec((ALIGN, NUM_CLASSES), lambda i: (0, 0)),
            pl.BlockSpec((1, HID), lambda i: (0, 0)),
            pl.BlockSpec((1, ALIGN), lambda i: (0, 0)),
            pl.BlockSpec((1, NUM_CLASSES), lambda i: (0, 0)),
        ],
        out_specs=[
            pl.BlockSpec((BR, ALIGN), lambda i: (i, 0)),
            pl.BlockSpec((BR, NUM_CLASSES), lambda i: (i, 0)),
            pl.BlockSpec((BR,), lambda i: (i,)),
        ],
        out_shape=[
            jax.ShapeDtypeStruct((N, ALIGN), jnp.float32),
            jax.ShapeDtypeStruct((N, NUM_CLASSES), jnp.float32),
            jax.ShapeDtypeStruct((N,), jnp.float32),
        ],
    )(acc2, deg0, deg1, zsem, W_pt, W_cls, b_gcn2, b_pt2, bcls2)


def kernel(x, edge_index, W_gcn, b_gcn, W_pt, b_pt, W_ps, b_ps, W_cls, b_cls):
    f32 = jnp.float32
    src2d = edge_index[0].reshape(E // CH, CH)
    dst2d = edge_index[1].reshape(E // CH, CH)

    deg0, deg1 = _sc_degree(dst2d, jnp.zeros((DEGPAD,), f32))

    y, zsem = _tc_pre(x, deg0, deg1, W_gcn, W_ps, b_ps.reshape(1, ALIGN))

    acc2 = _sc_scatter(y, src2d, dst2d, jnp.zeros((N, HID), f32))

    zt, logits, anomaly = _tc_post(
        acc2, deg0, deg1, zsem, W_pt, W_cls,
        b_gcn.reshape(1, HID), b_pt.reshape(1, ALIGN),
        b_cls.reshape(1, NUM_CLASSES))

    return (logits, anomaly, zt, zsem)


# speedup vs baseline: 64.3238x; 1.1199x over previous
import functools

import jax
import jax.numpy as jnp
from jax import lax
from jax.experimental import pallas as pl
from jax.experimental.pallas import tpu as pltpu
from jax.experimental.pallas import tpu_sc as plsc

N = 10000
E = 320000
IN_DIM = 128
HID = 64
ALIGN = 32
NUM_CLASSES = 7

NC = 2
NS = 16
NW = NC * NS

DEGPAD = 10240
DROWS = DEGPAD // NS
RPT = N // NS
CH = 125
CPW = 80
NB = 4

BR = 2048
GRID = (N + BR - 1) // BR


def _sc_mesh():
    return plsc.VectorSubcoreMesh(core_axis_name="c", subcore_axis_name="s")


def _deg_body(dst_hbm, zeros_hbm, out0_hbm, out1_hbm, idx_v, ones_v, acc_sh, isem, asem):
    c = lax.axis_index("c")
    s = lax.axis_index("s")
    w = s * NC + c
    rslice = pl.ds(s * DROWS, DROWS)
    for i in range(8):
        ones_v[pl.ds(i * 16, 16)] = jnp.ones((16,), jnp.float32)
    pltpu.async_copy(zeros_hbm.at[rslice], acc_sh.at[rslice], isem).wait()
    pltpu.sync_copy(dst_hbm.at[pl.ds(w * CPW, CPW)], idx_v)
    plsc.subcore_barrier()

    @pl.loop(0, CPW, step=NB)
    def _chunks(t):
        hs = [
            pltpu.async_copy(ones_v.at[pl.ds(0, CH)], acc_sh.at[idx_v.at[t + b]],
                             asem, add=True)
            for b in range(NB)
        ]
        for h in hs:
            h.wait()

    plsc.subcore_barrier()

    @pl.when(c == 0)
    def _():
        pltpu.sync_copy(acc_sh.at[rslice], out0_hbm.at[rslice])

    @pl.when(c != 0)
    def _():
        pltpu.sync_copy(acc_sh.at[rslice], out1_hbm.at[rslice])


def _sc_degree(dst2d, zeros1d):
    return pl.kernel(
        _deg_body,
        out_type=[jax.ShapeDtypeStruct((DEGPAD,), jnp.float32),
                  jax.ShapeDtypeStruct((DEGPAD,), jnp.float32)],
        mesh=_sc_mesh(),
        compiler_params=pltpu.CompilerParams(use_tc_tiling_on_sc=False),
        scratch_types=[
            pltpu.VMEM((CPW, CH), jnp.int32),
            pltpu.VMEM((128,), jnp.float32),
            pltpu.VMEM_SHARED((DEGPAD,), jnp.float32),
            pltpu.SemaphoreType.DMA,
            pltpu.SemaphoreType.DMA,
        ],
    )(dst2d, zeros1d)


def _scat_body(y_hbm, src_hbm, dst_hbm, zeros_hbm, out_hbm,
               src_v, dst_v, rows_v, acc_sh, isem, gsem0, gsem1, ssem0, ssem1):
    c = lax.axis_index("c")
    s = lax.axis_index("s")
    w = s * NC + c
    rslice = pl.ds(s * RPT, RPT)
    gsems = (gsem0, gsem1)
    ssems = (ssem0, ssem1)
    NG = CPW // NB

    def fire_g(g, par):
        for b in range(NB):
            pltpu.async_copy(y_hbm.at[src_v.at[NB * g + b]],
                             rows_v.at[par * NB + b], gsems[par])

    def drain_g(g, par):
        for b in range(NB):
            pltpu.make_async_copy(y_hbm.at[src_v.at[NB * g + b]],
                                  rows_v.at[par * NB + b], gsems[par]).wait()

    def fire_s(g, par):
        for b in range(NB):
            pltpu.async_copy(rows_v.at[par * NB + b],
                             acc_sh.at[dst_v.at[NB * g + b]], ssems[par],
                             add=True)

    def drain_s(g, par):
        for b in range(NB):
            pltpu.make_async_copy(rows_v.at[par * NB + b],
                                  acc_sh.at[dst_v.at[NB * g + b]],
                                  ssems[par]).wait()

    @pl.when(c == 0)
    def _():
        pltpu.async_copy(y_hbm.at[rslice], acc_sh.at[rslice], isem).wait()

    @pl.when(c != 0)
    def _():
        pltpu.async_copy(zeros_hbm.at[rslice], acc_sh.at[rslice], isem).wait()

    pltpu.sync_copy(src_hbm.at[pl.ds(w * CPW, CPW)], src_v)
    pltpu.sync_copy(dst_hbm.at[pl.ds(w * CPW, CPW)], dst_v)
    plsc.subcore_barrier()

    fire_g(0, 0)
    fire_g(1, 1)
    drain_g(0, 0)
    fire_s(0, 0)

    @pl.loop(2, NG, step=2)
    def _groups(g):
        drain_s(g - 2, 0)
        fire_g(g, 0)
        drain_g(g - 1, 1)
        fire_s(g - 1, 1)
        drain_s(g - 1, 1)
        fire_g(g + 1, 1)
        drain_g(g, 0)
        fire_s(g, 0)

    drain_s(NG - 2, 0)
    drain_g(NG - 1, 1)
    fire_s(NG - 1, 1)
    drain_s(NG - 1, 1)

    plsc.subcore_barrier()
    pltpu.sync_copy(acc_sh.at[rslice],
                    out_hbm.at[pl.ds(s * RPT, RPT), pl.ds(c * HID, HID)])


def _sc_scatter(y, src2d, dst2d, zeros2d):
    return pl.kernel(
        _scat_body,
        out_type=jax.ShapeDtypeStruct((N, 2 * HID), jnp.float32),
        mesh=_sc_mesh(),
        compiler_params=pltpu.CompilerParams(use_tc_tiling_on_sc=False),
        scratch_types=[
            pltpu.VMEM((CPW, CH), jnp.int32),
            pltpu.VMEM((CPW, CH), jnp.int32),
            pltpu.VMEM((2 * NB, CH, HID), jnp.float32),
            pltpu.VMEM_SHARED((N, HID), jnp.float32),
            pltpu.SemaphoreType.DMA,
            pltpu.SemaphoreType.DMA,
            pltpu.SemaphoreType.DMA,
            pltpu.SemaphoreType.DMA,
            pltpu.SemaphoreType.DMA,
        ],
    )(y, src2d, dst2d, zeros2d)


def _pre_body(x_ref, deg0_ref, deg1_ref, wg_ref, wps_ref, bps_ref,
              y_ref, zsem_ref):
    deg = deg0_ref[...] + deg1_ref[...] + 1.0
    dinv = lax.rsqrt(deg)
    xb = x_ref[...]
    xw = jnp.dot(xb, wg_ref[...], preferred_element_type=jnp.float32)
    y_ref[...] = xw * dinv[:, None]
    zs = (jnp.dot(xb, wps_ref[...], preferred_element_type=jnp.float32)
          + bps_ref[...])
    zsem_ref[...] = zs.T


def _tc_pre(xp, deg0, deg1, W_gcn, W_ps, b_ps2):
    return pl.pallas_call(
        _pre_body,
        grid=(GRID,),
        in_specs=[
            pl.BlockSpec((BR, IN_DIM), lambda i: (i, 0)),
            pl.BlockSpec((BR,), lambda i: (i,)),
            pl.BlockSpec((BR,), lambda i: (i,)),
            pl.BlockSpec((IN_DIM, HID), lambda i: (0, 0)),
            pl.BlockSpec((IN_DIM, ALIGN), lambda i: (0, 0)),
            pl.BlockSpec((1, ALIGN), lambda i: (0, 0)),
        ],
        out_specs=[
            pl.BlockSpec((BR, HID), lambda i: (i, 0)),
            pl.BlockSpec((ALIGN, BR), lambda i: (0, i)),
        ],
        out_shape=[
            jax.ShapeDtypeStruct((N, HID), jnp.float32),
            jax.ShapeDtypeStruct((ALIGN, N), jnp.float32),
        ],
    )(xp, deg0, deg1, W_gcn, W_ps, b_ps2)


def _post_body(acc_ref, deg0_ref, deg1_ref, zsem_ref, wpt_ref, wcls_ref,
               bg_ref, bpt_ref, bcls_ref, zt_ref, lg_ref, an_ref):
    deg = deg0_ref[...] + deg1_ref[...] + 1.0
    dinv = lax.rsqrt(deg)
    a2 = acc_ref[...]
    agg = (a2[:, :HID] + a2[:, HID:]) * dinv[:, None] + bg_ref[...]
    h = jnp.maximum(agg, 0.0)
    zt = jnp.dot(h, wpt_ref[...], preferred_element_type=jnp.float32) + bpt_ref[...]
    ztT = zt.T
    zt_ref[...] = ztT
    lg = (jnp.dot(zt, wcls_ref[...], preferred_element_type=jnp.float32)
          + bcls_ref[...])
    lg_ref[...] = lg.T
    diffT = ztT - zsem_ref[...]
    an_ref[...] = jnp.sqrt(jnp.sum(diffT * diffT, axis=0))


def _tc_post(acc2, deg0, deg1, zsem, W_pt, W_cls, b_gcn2, b_pt2, bcls2):
    return pl.pallas_call(
        _post_body,
        grid=(GRID,),
        in_specs=[
            pl.BlockSpec((BR, 2 * HID), lambda i: (i, 0)),
            pl.BlockSpec((BR,), lambda i: (i,)),
            pl.BlockSpec((BR,), lambda i: (i,)),
            pl.BlockSpec((ALIGN, BR), lambda i: (0, i)),
            pl.BlockSpec((HID, ALIGN), lambda i: (0, 0)),
            pl.BlockSpec((ALIGN, NUM_CLASSES), lambda i: (0, 0)),
            pl.BlockSpec((1, HID), lambda i: (0, 0)),
            pl.BlockSpec((1, ALIGN), lambda i: (0, 0)),
            pl.BlockSpec((1, NUM_CLASSES), lambda i: (0, 0)),
        ],
        out_specs=[
            pl.BlockSpec((ALIGN, BR), lambda i: (0, i)),
            pl.BlockSpec((NUM_CLASSES, BR), lambda i: (0, i)),
            pl.BlockSpec((BR,), lambda i: (i,)),
        ],
        out_shape=[
            jax.ShapeDtypeStruct((ALIGN, N), jnp.float32),
            jax.ShapeDtypeStruct((NUM_CLASSES, N), jnp.float32),
            jax.ShapeDtypeStruct((N,), jnp.float32),
        ],
    )(acc2, deg0, deg1, zsem, W_pt, W_cls, b_gcn2, b_pt2, bcls2)


def kernel(x, edge_index, W_gcn, b_gcn, W_pt, b_pt, W_ps, b_ps, W_cls, b_cls):
    f32 = jnp.float32
    src2d = edge_index[0].reshape(E // CH, CH)
    dst2d = edge_index[1].reshape(E // CH, CH)

    deg0, deg1 = _sc_degree(dst2d, jnp.zeros((DEGPAD,), f32))

    y, zsemT = _tc_pre(x, deg0, deg1, W_gcn, W_ps, b_ps.reshape(1, ALIGN))

    acc2 = _sc_scatter(y, src2d, dst2d, jnp.zeros((N, HID), f32))

    ztT, lgT, anomaly = _tc_post(
        acc2, deg0, deg1, zsemT, W_pt, W_cls,
        b_gcn.reshape(1, HID), b_pt.reshape(1, ALIGN),
        b_cls.reshape(1, NUM_CLASSES))

    return (lgT.T, anomaly, ztT.T, zsemT.T)
